# Initial kernel scaffold; baseline (speedup 1.0000x reference)
import jax
import jax.numpy as jnp
from jax.experimental import pallas as pl


def kernel(x, edge_index, W1, b1, W2, b2):
    raise NotImplementedError("write your pallas kernel here")


# speedup vs baseline: 26.2617x; 26.2617x over previous
import functools

import jax
import jax.numpy as jnp
from jax import lax
from jax.experimental import pallas as pl
from jax.experimental.pallas import tpu as pltpu
from jax.experimental.pallas import tpu_sc as plsc

N_NODES = 10000
R_PAD = 10240
NC, NS = 2, 16
NW = NC * NS
CHUNK = 128
ROWS_PT = R_PAD // NS
OUT_PT = N_NODES // NS
RB = 1000


def _fill1d(ref, n, val):
    def body(i, _):
        ref[pl.ds(i * 16, 16)] = jnp.full((16,), val, ref.dtype)
        return 0
    lax.fori_loop(0, n // 16, body, 0)


def _fill2d(ref, rows, cols, val):
    def body(i, _):
        r = i // (cols // 16)
        k = i % (cols // 16)
        ref[r, pl.ds(k * 16, 16)] = jnp.full((16,), val, ref.dtype)
        return 0
    lax.fori_loop(0, rows * (cols // 16), body, 0)


def _deg_body(cpwd, dst3d, deg_out, dstv, ones_v, stage, deg_sh):
    c = lax.axis_index("c")
    s = lax.axis_index("s")
    wid = c * NS + s
    _fill1d(stage, ROWS_PT, 0.0)
    pltpu.sync_copy(stage, deg_sh.at[pl.ds(s * ROWS_PT, ROWS_PT)])
    plsc.subcore_barrier()
    _fill1d(ones_v, CHUNK, 1.0)
    pltpu.sync_copy(dst3d.at[wid], dstv)

    def body(j, _):
        pltpu.sync_copy(ones_v, deg_sh.at[dstv.at[j]], add=True)
        return 0
    lax.fori_loop(0, cpwd, body, 0)
    plsc.subcore_barrier()
    pltpu.sync_copy(deg_sh.at[pl.ds(s * ROWS_PT, ROWS_PT)], stage)
    pltpu.sync_copy(stage, deg_out.at[c, 0, pl.ds(s * ROWS_PT, ROWS_PT)])


def _sc_deg(dst3d_deg, cpwd):
    kern = pl.kernel(
        functools.partial(_deg_body, cpwd),
        out_type=jax.ShapeDtypeStruct((NC, 1, R_PAD), jnp.float32),
        mesh=plsc.VectorSubcoreMesh(core_axis_name="c", subcore_axis_name="s"),
        compiler_params=pltpu.CompilerParams(use_tc_tiling_on_sc=False),
        scratch_types=[
            pltpu.VMEM((cpwd, CHUNK), jnp.int32),
            pltpu.VMEM((CHUNK,), jnp.float32),
            pltpu.VMEM((ROWS_PT,), jnp.float32),
            pltpu.VMEM_SHARED((R_PAD,), jnp.float32),
        ],
    )
    return kern(dst3d_deg)


def _agg_body(cpt, d, staged, hs2, src3d, dst3d, out,
              srcv, dstv, buf0, buf1, acc_sh, sem0, sem1, *maybe_tab):
    c = lax.axis_index("c")
    s = lax.axis_index("s")
    _fill2d(buf0, CHUNK, d, 0.0)
    for r in range(ROWS_PT // CHUNK):
        pltpu.sync_copy(buf0, acc_sh.at[pl.ds(s * ROWS_PT + r * CHUNK, CHUNK)])
    if staged:
        tab = maybe_tab[0]
        pltpu.sync_copy(hs2.at[c, pl.ds(s * OUT_PT, OUT_PT)],
                        tab.at[pl.ds(s * OUT_PT, OUT_PT)])
    else:
        tab = hs2.at[c]
    plsc.subcore_barrier()
    pltpu.sync_copy(src3d.at[s], srcv)
    pltpu.sync_copy(dst3d.at[s], dstv)

    pltpu.async_copy(tab.at[srcv.at[0]], buf0, sem0)

    def pair(jj, _):
        j0 = 2 * jj
        j1 = j0 + 1
        jn = jnp.where(j0 + 2 < cpt, j0 + 2, 0)
        pltpu.make_async_copy(tab.at[srcv.at[0]], buf0, sem0).wait()
        pltpu.async_copy(tab.at[srcv.at[j1]], buf1, sem1)
        pltpu.sync_copy(buf0, acc_sh.at[dstv.at[j0]], add=True)
        pltpu.make_async_copy(tab.at[srcv.at[0]], buf1, sem1).wait()
        pltpu.async_copy(tab.at[srcv.at[jn]], buf0, sem0)
        pltpu.sync_copy(buf1, acc_sh.at[dstv.at[j1]], add=True)
        return 0
    lax.fori_loop(0, cpt // 2, pair, 0)
    pltpu.make_async_copy(tab.at[srcv.at[0]], buf0, sem0).wait()

    plsc.subcore_barrier()
    for r0, nr in ((0, CHUNK), (CHUNK, CHUNK), (2 * CHUNK, CHUNK),
                   (3 * CHUNK, CHUNK), (4 * CHUNK, OUT_PT - 4 * CHUNK)):
        row0 = s * OUT_PT + r0
        pltpu.sync_copy(acc_sh.at[pl.ds(row0, nr)], buf0.at[pl.ds(0, nr)])
        pltpu.sync_copy(buf0.at[pl.ds(0, nr)], out.at[c, pl.ds(row0, nr)])


def _sc_agg(hs2, src3d, dst3d, cpt, d, staged):
    scratch = [
        pltpu.VMEM((cpt, CHUNK), jnp.int32),
        pltpu.VMEM((cpt, CHUNK), jnp.int32),
        pltpu.VMEM((CHUNK, d), jnp.float32),
        pltpu.VMEM((CHUNK, d), jnp.float32),
        pltpu.VMEM_SHARED((R_PAD, d), jnp.float32),
        pltpu.SemaphoreType.DMA,
        pltpu.SemaphoreType.DMA,
    ]
    if staged:
        scratch.append(pltpu.VMEM_SHARED((N_NODES, d), jnp.float32))
    kern = pl.kernel(
        functools.partial(_agg_body, cpt, d, staged),
        out_type=jax.ShapeDtypeStruct((NC, N_NODES, d), jnp.float32),
        mesh=plsc.VectorSubcoreMesh(core_axis_name="c", subcore_axis_name="s"),
        compiler_params=pltpu.CompilerParams(use_tc_tiling_on_sc=False),
        scratch_types=scratch,
    )
    return kern(hs2, src3d, dst3d)


def _mm1_body(x_ref, w_ref, deg_ref, o_ref):
    dinv = lax.rsqrt(deg_ref[0] + deg_ref[1] + 1.0)
    h = jnp.dot(x_ref[...], w_ref[...], preferred_element_type=jnp.float32)
    hs = h * dinv
    o_ref[0] = hs[:, :64]
    o_ref[1] = hs[:, 64:]


def _mm2_body(p_ref, h_ref, deg_ref, b_ref, w_ref, o_ref):
    dinv = lax.rsqrt(deg_ref[0] + deg_ref[1] + 1.0)
    agg = jnp.concatenate([p_ref[0] + h_ref[0], p_ref[1] + h_ref[1]], axis=1)
    h1 = jnp.maximum(agg * dinv + b_ref[...], 0.0)
    g = jnp.dot(h1, w_ref[...], preferred_element_type=jnp.float32) * dinv
    o_ref[0] = g[:, :32]
    o_ref[1] = g[:, 32:]


def _fin_body(p_ref, g_ref, deg_ref, b_ref, o_ref):
    dinv = lax.rsqrt(deg_ref[0] + deg_ref[1] + 1.0)
    agg = jnp.concatenate([p_ref[0] + g_ref[0], p_ref[1] + g_ref[1]], axis=1)
    o_ref[...] = agg * dinv + b_ref[...]


def _tc_mm1(x, w1, deg3d):
    return pl.pallas_call(
        _mm1_body,
        grid=(N_NODES // RB,),
        in_specs=[
            pl.BlockSpec((RB, 128), lambda i: (i, 0)),
            pl.BlockSpec((128, 128), lambda i: (0, 0)),
            pl.BlockSpec((NC, RB, 1), lambda i: (0, i, 0)),
        ],
        out_specs=pl.BlockSpec((NC, RB, 64), lambda i: (0, i, 0)),
        out_shape=jax.ShapeDtypeStruct((NC, N_NODES, 64), jnp.float32),
    )(x, w1, deg3d)


def _tc_mm2(part1, hs2, deg3d, b1, w2):
    return pl.pallas_call(
        _mm2_body,
        grid=(N_NODES // RB,),
        in_specs=[
            pl.BlockSpec((NC, RB, 64), lambda i: (0, i, 0)),
            pl.BlockSpec((NC, RB, 64), lambda i: (0, i, 0)),
            pl.BlockSpec((NC, RB, 1), lambda i: (0, i, 0)),
            pl.BlockSpec((1, 128), lambda i: (0, 0)),
            pl.BlockSpec((128, 64), lambda i: (0, 0)),
        ],
        out_specs=pl.BlockSpec((NC, RB, 32), lambda i: (0, i, 0)),
        out_shape=jax.ShapeDtypeStruct((NC, N_NODES, 32), jnp.float32),
    )(part1, hs2, deg3d, b1, w2)


def _tc_fin(part2, gs2, deg3d, b2):
    return pl.pallas_call(
        _fin_body,
        grid=(N_NODES // RB,),
        in_specs=[
            pl.BlockSpec((NC, RB, 32), lambda i: (0, i, 0)),
            pl.BlockSpec((NC, RB, 32), lambda i: (0, i, 0)),
            pl.BlockSpec((NC, RB, 1), lambda i: (0, i, 0)),
            pl.BlockSpec((1, 64), lambda i: (0, 0)),
        ],
        out_specs=pl.BlockSpec((RB, 64), lambda i: (i, 0)),
        out_shape=jax.ShapeDtypeStruct((N_NODES, 64), jnp.float32),
    )(part2, gs2, deg3d, b2)


def kernel(x, edge_index, W1, b1, W2, b2):
    n = x.shape[0]
    e = edge_index.shape[1]
    cpt = -(-e // (NS * CHUNK))
    cpt += cpt % 2
    e_padded = NS * cpt * CHUNK
    cpwd = cpt // 2
    npad = e_padded - e

    pad_src = jnp.arange(npad, dtype=jnp.int32) % n
    pad_dst = n + jnp.arange(npad, dtype=jnp.int32) % (R_PAD - n)
    src = jnp.concatenate([edge_index[0], pad_src])
    dst = jnp.concatenate([edge_index[1], pad_dst])
    src3d = src.reshape(NS, cpt, CHUNK)
    dst3d = dst.reshape(NS, cpt, CHUNK)
    dst3d_deg = dst.reshape(NW, cpwd, CHUNK)

    deg3d = _sc_deg(dst3d_deg, cpwd).reshape(NC, R_PAD, 1)
    hs2 = _tc_mm1(x, W1, deg3d)
    part1 = _sc_agg(hs2, src3d, dst3d, cpt, 64, False)
    gs2 = _tc_mm2(part1, hs2, deg3d, b1.reshape(1, 128), W2)
    part2 = _sc_agg(gs2, src3d, dst3d, cpt, 32, True)
    out = _tc_fin(part2, gs2, deg3d, b2.reshape(1, 64))
    return out


# speedup vs baseline: 31.7560x; 1.2092x over previous
import functools

import jax
import jax.numpy as jnp
from jax import lax
from jax.experimental import pallas as pl
from jax.experimental.pallas import tpu as pltpu
from jax.experimental.pallas import tpu_sc as plsc

N_NODES = 10000
R_PAD = 10240
NC, NS = 2, 16
NW = NC * NS
CHUNK = 128
ROWS_PT = R_PAD // NS
OUT_PT = N_NODES // NS
RB = 1000


def _lanes(dtype):
    return 32 if jnp.dtype(dtype).itemsize == 2 else 16


def _fill1d(ref, n, val):
    ln = _lanes(ref.dtype)
    def body(i, _):
        ref[pl.ds(i * ln, ln)] = jnp.full((ln,), val, ref.dtype)
        return 0
    lax.fori_loop(0, n // ln, body, 0)


def _fill2d(ref, rows, cols, val):
    ln = _lanes(ref.dtype)
    def body(i, _):
        r = i // (cols // ln)
        k = i % (cols // ln)
        ref[r, pl.ds(k * ln, ln)] = jnp.full((ln,), val, ref.dtype)
        return 0
    lax.fori_loop(0, rows * (cols // ln), body, 0)


def _deg_body(cpwd, dst3d, deg_out, dstv, ones_v, stage, deg_sh):
    c = lax.axis_index("c")
    s = lax.axis_index("s")
    wid = c * NS + s
    _fill1d(stage, ROWS_PT, 0.0)
    pltpu.sync_copy(stage, deg_sh.at[pl.ds(s * ROWS_PT, ROWS_PT)])
    plsc.subcore_barrier()
    _fill1d(ones_v, CHUNK, 1.0)
    pltpu.sync_copy(dst3d.at[wid], dstv)

    def body(j, _):
        pltpu.sync_copy(ones_v, deg_sh.at[dstv.at[j]], add=True)
        return 0
    lax.fori_loop(0, cpwd, body, 0)
    plsc.subcore_barrier()
    pltpu.sync_copy(deg_sh.at[pl.ds(s * ROWS_PT, ROWS_PT)], stage)
    pltpu.sync_copy(stage, deg_out.at[c, 0, pl.ds(s * ROWS_PT, ROWS_PT)])


def _sc_deg(dst3d_deg, cpwd):
    kern = pl.kernel(
        functools.partial(_deg_body, cpwd),
        out_type=jax.ShapeDtypeStruct((NC, 1, R_PAD), jnp.float32),
        mesh=plsc.VectorSubcoreMesh(core_axis_name="c", subcore_axis_name="s"),
        compiler_params=pltpu.CompilerParams(use_tc_tiling_on_sc=False),
        scratch_types=[
            pltpu.VMEM((cpwd, CHUNK), jnp.int32),
            pltpu.VMEM((CHUNK,), jnp.float32),
            pltpu.VMEM((ROWS_PT,), jnp.float32),
            pltpu.VMEM_SHARED((R_PAD,), jnp.float32),
        ],
    )
    return kern(dst3d_deg)


def _agg_body(cpt, d, staged, hs2, src3d, dst3d, out,
              srcv, dstv, buf0, buf1, acc_sh, sem0, sem1, *maybe_tab):
    c = lax.axis_index("c")
    s = lax.axis_index("s")
    _fill2d(buf0, CHUNK, d, 0.0)
    for r in range(ROWS_PT // CHUNK):
        pltpu.sync_copy(buf0, acc_sh.at[pl.ds(s * ROWS_PT + r * CHUNK, CHUNK)])
    if staged:
        tab = maybe_tab[0]
        pltpu.sync_copy(hs2.at[c, pl.ds(s * OUT_PT, OUT_PT)],
                        tab.at[pl.ds(s * OUT_PT, OUT_PT)])
    else:
        tab = hs2.at[c]
    plsc.subcore_barrier()
    pltpu.sync_copy(src3d.at[s], srcv)
    pltpu.sync_copy(dst3d.at[s], dstv)

    pltpu.async_copy(tab.at[srcv.at[0]], buf0, sem0)

    def pair(jj, _):
        j0 = 2 * jj
        j1 = j0 + 1
        jn = jnp.where(j0 + 2 < cpt, j0 + 2, 0)
        pltpu.make_async_copy(tab.at[srcv.at[0]], buf0, sem0).wait()
        pltpu.async_copy(tab.at[srcv.at[j1]], buf1, sem1)
        pltpu.sync_copy(buf0, acc_sh.at[dstv.at[j0]], add=True)
        pltpu.make_async_copy(tab.at[srcv.at[0]], buf1, sem1).wait()
        pltpu.async_copy(tab.at[srcv.at[jn]], buf0, sem0)
        pltpu.sync_copy(buf1, acc_sh.at[dstv.at[j1]], add=True)
        return 0
    lax.fori_loop(0, cpt // 2, pair, 0)
    pltpu.make_async_copy(tab.at[srcv.at[0]], buf0, sem0).wait()

    plsc.subcore_barrier()
    for r0, nr in ((0, CHUNK), (CHUNK, CHUNK), (2 * CHUNK, CHUNK),
                   (3 * CHUNK, CHUNK), (4 * CHUNK, OUT_PT - 4 * CHUNK)):
        row0 = s * OUT_PT + r0
        pltpu.sync_copy(acc_sh.at[pl.ds(row0, nr)], buf0.at[pl.ds(0, nr)])
        pltpu.sync_copy(buf0.at[pl.ds(0, nr)], out.at[c, pl.ds(row0, nr)])


def _sc_agg(hs2, src3d, dst3d, cpt, d, staged):
    scratch = [
        pltpu.VMEM((cpt, CHUNK), jnp.int32),
        pltpu.VMEM((cpt, CHUNK), jnp.int32),
        pltpu.VMEM((CHUNK, d), jnp.bfloat16),
        pltpu.VMEM((CHUNK, d), jnp.bfloat16),
        pltpu.VMEM_SHARED((R_PAD, d), jnp.bfloat16),
        pltpu.SemaphoreType.DMA,
        pltpu.SemaphoreType.DMA,
    ]
    if staged:
        scratch.append(pltpu.VMEM_SHARED((N_NODES, d), jnp.bfloat16))
    kern = pl.kernel(
        functools.partial(_agg_body, cpt, d, staged),
        out_type=jax.ShapeDtypeStruct((NC, N_NODES, d), jnp.bfloat16),
        mesh=plsc.VectorSubcoreMesh(core_axis_name="c", subcore_axis_name="s"),
        compiler_params=pltpu.CompilerParams(use_tc_tiling_on_sc=False),
        scratch_types=scratch,
    )
    return kern(hs2, src3d, dst3d)


def _mm1_body(x_ref, w_ref, deg_ref, o_ref):
    dinv = lax.rsqrt(deg_ref[0] + deg_ref[1] + 1.0)
    h = jnp.dot(x_ref[...], w_ref[...], preferred_element_type=jnp.float32)
    hs = (h * dinv).astype(jnp.bfloat16)
    o_ref[0] = hs[:, :64]
    o_ref[1] = hs[:, 64:]


def _mm2_body(p_ref, h_ref, deg_ref, b_ref, w_ref, o_ref):
    dinv = lax.rsqrt(deg_ref[0] + deg_ref[1] + 1.0)
    agg = jnp.concatenate([p_ref[0] + h_ref[0], p_ref[1] + h_ref[1]],
                          axis=1).astype(jnp.float32)
    h1 = jnp.maximum(agg * dinv + b_ref[...], 0.0)
    g = ((jnp.dot(h1, w_ref[...], preferred_element_type=jnp.float32) * dinv)
         .astype(jnp.bfloat16))
    o_ref[0] = g[:, :32]
    o_ref[1] = g[:, 32:]


def _fin_body(p_ref, g_ref, deg_ref, b_ref, o_ref):
    dinv = lax.rsqrt(deg_ref[0] + deg_ref[1] + 1.0)
    agg = jnp.concatenate([p_ref[0] + g_ref[0], p_ref[1] + g_ref[1]],
                          axis=1).astype(jnp.float32)
    o_ref[...] = agg * dinv + b_ref[...]


def _tc_mm1(x, w1, deg3d):
    return pl.pallas_call(
        _mm1_body,
        grid=(N_NODES // RB,),
        in_specs=[
            pl.BlockSpec((RB, 128), lambda i: (i, 0)),
            pl.BlockSpec((128, 128), lambda i: (0, 0)),
            pl.BlockSpec((NC, RB, 1), lambda i: (0, i, 0)),
        ],
        out_specs=pl.BlockSpec((NC, RB, 64), lambda i: (0, i, 0)),
        out_shape=jax.ShapeDtypeStruct((NC, N_NODES, 64), jnp.bfloat16),
    )(x, w1, deg3d)


def _tc_mm2(part1, hs2, deg3d, b1, w2):
    return pl.pallas_call(
        _mm2_body,
        grid=(N_NODES // RB,),
        in_specs=[
            pl.BlockSpec((NC, RB, 64), lambda i: (0, i, 0)),
            pl.BlockSpec((NC, RB, 64), lambda i: (0, i, 0)),
            pl.BlockSpec((NC, RB, 1), lambda i: (0, i, 0)),
            pl.BlockSpec((1, 128), lambda i: (0, 0)),
            pl.BlockSpec((128, 64), lambda i: (0, 0)),
        ],
        out_specs=pl.BlockSpec((NC, RB, 32), lambda i: (0, i, 0)),
        out_shape=jax.ShapeDtypeStruct((NC, N_NODES, 32), jnp.bfloat16),
    )(part1, hs2, deg3d, b1, w2)


def _tc_fin(part2, gs2, deg3d, b2):
    return pl.pallas_call(
        _fin_body,
        grid=(N_NODES // RB,),
        in_specs=[
            pl.BlockSpec((NC, RB, 32), lambda i: (0, i, 0)),
            pl.BlockSpec((NC, RB, 32), lambda i: (0, i, 0)),
            pl.BlockSpec((NC, RB, 1), lambda i: (0, i, 0)),
            pl.BlockSpec((1, 64), lambda i: (0, 0)),
        ],
        out_specs=pl.BlockSpec((RB, 64), lambda i: (i, 0)),
        out_shape=jax.ShapeDtypeStruct((N_NODES, 64), jnp.float32),
    )(part2, gs2, deg3d, b2)


def kernel(x, edge_index, W1, b1, W2, b2):
    n = x.shape[0]
    e = edge_index.shape[1]
    cpt = -(-e // (NS * CHUNK))
    cpt += cpt % 2
    e_padded = NS * cpt * CHUNK
    cpwd = cpt // 2
    npad = e_padded - e

    pad_src = jnp.arange(npad, dtype=jnp.int32) % n
    pad_dst = n + jnp.arange(npad, dtype=jnp.int32) % (R_PAD - n)
    src = jnp.concatenate([edge_index[0], pad_src])
    dst = jnp.concatenate([edge_index[1], pad_dst])
    src3d = src.reshape(NS, cpt, CHUNK)
    dst3d = dst.reshape(NS, cpt, CHUNK)
    dst3d_deg = dst.reshape(NW, cpwd, CHUNK)

    deg3d = _sc_deg(dst3d_deg, cpwd).reshape(NC, R_PAD, 1)
    hs2 = _tc_mm1(x, W1, deg3d)
    part1 = _sc_agg(hs2, src3d, dst3d, cpt, 64, False)
    gs2 = _tc_mm2(part1, hs2, deg3d, b1.reshape(1, 128), W2)
    part2 = _sc_agg(gs2, src3d, dst3d, cpt, 32, True)
    out = _tc_fin(part2, gs2, deg3d, b2.reshape(1, 64))
    return out


# speedup vs baseline: 38.7285x; 1.2196x over previous
import functools

import jax
import jax.numpy as jnp
from jax import lax
from jax.experimental import pallas as pl
from jax.experimental.pallas import tpu as pltpu
from jax.experimental.pallas import tpu_sc as plsc

N_NODES = 10000
R_PAD = 10240
NC, NS = 2, 16
NW = NC * NS
CHUNK = 128
ROWS_PT = R_PAD // NS
OUT_PT = N_NODES // NS
RB = 1000


def _lanes(dtype):
    return 32 if jnp.dtype(dtype).itemsize == 2 else 16


def _fill1d(ref, n, val):
    ln = _lanes(ref.dtype)
    def body(i, _):
        ref[pl.ds(i * ln, ln)] = jnp.full((ln,), val, ref.dtype)
        return 0
    lax.fori_loop(0, n // ln, body, 0)


def _fill2d(ref, rows, cols, val):
    ln = _lanes(ref.dtype)
    def body(i, _):
        r = i // (cols // ln)
        k = i % (cols // ln)
        ref[r, pl.ds(k * ln, ln)] = jnp.full((ln,), val, ref.dtype)
        return 0
    lax.fori_loop(0, rows * (cols // ln), body, 0)


def _deg_body(cpwd, dst3d, deg_out, dstv, ones_v, stage, deg_sh):
    c = lax.axis_index("c")
    s = lax.axis_index("s")
    wid = c * NS + s
    _fill1d(stage, ROWS_PT, 0.0)
    pltpu.sync_copy(stage, deg_sh.at[pl.ds(s * ROWS_PT, ROWS_PT)])
    plsc.subcore_barrier()
    _fill1d(ones_v, CHUNK, 1.0)
    pltpu.sync_copy(dst3d.at[wid], dstv)

    def body(j, _):
        pltpu.sync_copy(ones_v, deg_sh.at[dstv.at[j]], add=True)
        return 0
    lax.fori_loop(0, cpwd, body, 0)
    plsc.subcore_barrier()
    pltpu.sync_copy(deg_sh.at[pl.ds(s * ROWS_PT, ROWS_PT)], stage)
    pltpu.sync_copy(stage, deg_out.at[c, 0, pl.ds(s * ROWS_PT, ROWS_PT)])


def _sc_deg(dst3d_deg, cpwd):
    kern = pl.kernel(
        functools.partial(_deg_body, cpwd),
        out_type=jax.ShapeDtypeStruct((NC, 1, R_PAD), jnp.float32),
        mesh=plsc.VectorSubcoreMesh(core_axis_name="c", subcore_axis_name="s"),
        compiler_params=pltpu.CompilerParams(use_tc_tiling_on_sc=False),
        scratch_types=[
            pltpu.VMEM((cpwd, CHUNK), jnp.int32),
            pltpu.VMEM((CHUNK,), jnp.float32),
            pltpu.VMEM((ROWS_PT,), jnp.float32),
            pltpu.VMEM_SHARED((R_PAD,), jnp.float32),
        ],
    )
    return kern(dst3d_deg)


def _agg_body(cpt, d, staged, hs2, src3d, dst3d, out,
              srcv, dstv, buf0, buf1, acc_sh, sem0, sem1, *maybe_tab):
    c = lax.axis_index("c")
    s = lax.axis_index("s")
    _fill2d(buf0, CHUNK, d, 0.0)
    for r in range(ROWS_PT // CHUNK):
        pltpu.sync_copy(buf0, acc_sh.at[pl.ds(s * ROWS_PT + r * CHUNK, CHUNK)])
    if staged:
        tab = maybe_tab[0]
        pltpu.sync_copy(hs2.at[c, pl.ds(s * OUT_PT, OUT_PT)],
                        tab.at[pl.ds(s * OUT_PT, OUT_PT)])
    else:
        tab = hs2.at[c]
    plsc.subcore_barrier()
    pltpu.sync_copy(src3d.at[s], srcv)
    pltpu.sync_copy(dst3d.at[s], dstv)

    pltpu.async_copy(tab.at[srcv.at[0]], buf0, sem0)

    def pair(jj, _):
        j0 = 2 * jj
        j1 = j0 + 1
        jn = jnp.where(j0 + 2 < cpt, j0 + 2, 0)
        pltpu.make_async_copy(tab.at[srcv.at[0]], buf0, sem0).wait()
        pltpu.async_copy(tab.at[srcv.at[j1]], buf1, sem1)
        pltpu.sync_copy(buf0, acc_sh.at[dstv.at[j0]], add=True)
        pltpu.make_async_copy(tab.at[srcv.at[0]], buf1, sem1).wait()
        pltpu.async_copy(tab.at[srcv.at[jn]], buf0, sem0)
        pltpu.sync_copy(buf1, acc_sh.at[dstv.at[j1]], add=True)
        return 0
    lax.fori_loop(0, cpt // 2, pair, 0)
    pltpu.make_async_copy(tab.at[srcv.at[0]], buf0, sem0).wait()

    plsc.subcore_barrier()
    for r0, nr in ((0, CHUNK), (CHUNK, CHUNK), (2 * CHUNK, CHUNK),
                   (3 * CHUNK, CHUNK), (4 * CHUNK, OUT_PT - 4 * CHUNK)):
        row0 = s * OUT_PT + r0
        pltpu.sync_copy(acc_sh.at[pl.ds(row0, nr)], buf0.at[pl.ds(0, nr)])
        pltpu.sync_copy(buf0.at[pl.ds(0, nr)], out.at[c, pl.ds(row0, nr)])


def _sc_agg(hs2, src3d, dst3d, cpt, d, staged):
    scratch = [
        pltpu.VMEM((cpt, CHUNK), jnp.int32),
        pltpu.VMEM((cpt, CHUNK), jnp.int32),
        pltpu.VMEM((CHUNK, d), jnp.bfloat16),
        pltpu.VMEM((CHUNK, d), jnp.bfloat16),
        pltpu.VMEM_SHARED((R_PAD, d), jnp.bfloat16),
        pltpu.SemaphoreType.DMA,
        pltpu.SemaphoreType.DMA,
    ]
    if staged:
        scratch.append(pltpu.VMEM_SHARED((N_NODES, d), jnp.bfloat16))
    kern = pl.kernel(
        functools.partial(_agg_body, cpt, d, staged),
        out_type=jax.ShapeDtypeStruct((NC, N_NODES, d), jnp.bfloat16),
        mesh=plsc.VectorSubcoreMesh(core_axis_name="c", subcore_axis_name="s"),
        compiler_params=pltpu.CompilerParams(use_tc_tiling_on_sc=False),
        scratch_types=scratch,
    )
    return kern(hs2, src3d, dst3d)


def _mm1_body(x_ref, w_ref, deg_ref, o_ref):
    dinv = lax.rsqrt(deg_ref[0] + deg_ref[1] + 1.0)
    h = jnp.dot(x_ref[...], w_ref[...], preferred_element_type=jnp.float32)
    hs = (h * dinv).astype(jnp.bfloat16)
    o_ref[0] = hs[:, :64]
    o_ref[1] = hs[:, 64:]


def _mm2_body(p_ref, h_ref, deg_ref, b_ref, w_ref, o_ref):
    dinv = lax.rsqrt(deg_ref[0] + deg_ref[1] + 1.0)
    agg = jnp.concatenate([p_ref[0] + h_ref[0], p_ref[1] + h_ref[1]],
                          axis=1).astype(jnp.float32)
    h1 = jnp.maximum(agg * dinv + b_ref[...], 0.0)
    g = ((jnp.dot(h1, w_ref[...], preferred_element_type=jnp.float32) * dinv)
         .astype(jnp.bfloat16))
    o_ref[0] = g[:, :32]
    o_ref[1] = g[:, 32:]


def _fin_body(p_ref, g_ref, deg_ref, b_ref, o_ref):
    dinv = lax.rsqrt(deg_ref[0] + deg_ref[1] + 1.0)
    agg = jnp.concatenate([p_ref[0] + g_ref[0], p_ref[1] + g_ref[1]],
                          axis=1).astype(jnp.float32)
    o_ref[...] = agg * dinv + b_ref[...]


def _tc_mm1(x, w1, deg3d):
    return pl.pallas_call(
        _mm1_body,
        grid=(N_NODES // RB,),
        in_specs=[
            pl.BlockSpec((RB, 128), lambda i: (i, 0)),
            pl.BlockSpec((128, 128), lambda i: (0, 0)),
            pl.BlockSpec((NC, RB, 1), lambda i: (0, i, 0)),
        ],
        out_specs=pl.BlockSpec((NC, RB, 64), lambda i: (0, i, 0)),
        out_shape=jax.ShapeDtypeStruct((NC, N_NODES, 64), jnp.bfloat16),
    )(x, w1, deg3d)


def _tc_mm2(part1, hs2, deg3d, b1, w2):
    return pl.pallas_call(
        _mm2_body,
        grid=(N_NODES // RB,),
        in_specs=[
            pl.BlockSpec((NC, RB, 64), lambda i: (0, i, 0)),
            pl.BlockSpec((NC, RB, 64), lambda i: (0, i, 0)),
            pl.BlockSpec((NC, RB, 1), lambda i: (0, i, 0)),
            pl.BlockSpec((1, 128), lambda i: (0, 0)),
            pl.BlockSpec((128, 64), lambda i: (0, 0)),
        ],
        out_specs=pl.BlockSpec((NC, RB, 32), lambda i: (0, i, 0)),
        out_shape=jax.ShapeDtypeStruct((NC, N_NODES, 32), jnp.bfloat16),
    )(part1, hs2, deg3d, b1, w2)


def _tc_fin(part2, gs2, deg3d, b2):
    return pl.pallas_call(
        _fin_body,
        grid=(N_NODES // RB,),
        in_specs=[
            pl.BlockSpec((NC, RB, 32), lambda i: (0, i, 0)),
            pl.BlockSpec((NC, RB, 32), lambda i: (0, i, 0)),
            pl.BlockSpec((NC, RB, 1), lambda i: (0, i, 0)),
            pl.BlockSpec((1, 64), lambda i: (0, 0)),
        ],
        out_specs=pl.BlockSpec((RB, 64), lambda i: (i, 0)),
        out_shape=jax.ShapeDtypeStruct((N_NODES, 64), jnp.float32),
    )(part2, gs2, deg3d, b2)


def kernel(x, edge_index, W1, b1, W2, b2):
    n = x.shape[0]
    e = edge_index.shape[1]
    cpt = -(-e // (NS * CHUNK))
    cpt += cpt % 2
    e_padded = NS * cpt * CHUNK
    cpwd = cpt // 2
    npad = e_padded - e

    pad_src = jnp.arange(npad, dtype=jnp.int32) % n
    pad_dst = n + jnp.arange(npad, dtype=jnp.int32) % (R_PAD - n)
    src = jnp.concatenate([edge_index[0], pad_src])
    dst = jnp.concatenate([edge_index[1], pad_dst])
    src3d = src.reshape(NS, cpt, CHUNK)
    dst3d = dst.reshape(NS, cpt, CHUNK)
    dst3d_deg = dst.reshape(NW, cpwd, CHUNK)

    deg3d = _sc_deg(dst3d_deg, cpwd).reshape(NC, R_PAD, 1)
    hs2 = _tc_mm1(x, W1, deg3d)
    part1 = _sc_agg(hs2, src3d, dst3d, cpt, 64, True)
    gs2 = _tc_mm2(part1, hs2, deg3d, b1.reshape(1, 128), W2)
    part2 = _sc_agg(gs2, src3d, dst3d, cpt, 32, True)
    out = _tc_fin(part2, gs2, deg3d, b2.reshape(1, 64))
    return out


# speedup vs baseline: 38.9610x; 1.0060x over previous
import functools

import jax
import jax.numpy as jnp
from jax import lax
from jax.experimental import pallas as pl
from jax.experimental.pallas import tpu as pltpu
from jax.experimental.pallas import tpu_sc as plsc

N_NODES = 10000
R_PAD = 10240
NC, NS = 2, 16
NW = NC * NS
CHUNK = 128
ROWS_PT = R_PAD // NS
OUT_PT = N_NODES // NS
RB = 1000


def _lanes(dtype):
    return 32 if jnp.dtype(dtype).itemsize == 2 else 16


def _fill1d(ref, n, val):
    ln = _lanes(ref.dtype)
    def body(i, _):
        ref[pl.ds(i * ln, ln)] = jnp.full((ln,), val, ref.dtype)
        return 0
    lax.fori_loop(0, n // ln, body, 0)


def _fill2d(ref, rows, cols, val):
    ln = _lanes(ref.dtype)
    def body(i, _):
        r = i // (cols // ln)
        k = i % (cols // ln)
        ref[r, pl.ds(k * ln, ln)] = jnp.full((ln,), val, ref.dtype)
        return 0
    lax.fori_loop(0, rows * (cols // ln), body, 0)


def _deg_body(cpwd, dst3d, deg_out, dstv, ones_v, stage, deg_sh):
    c = lax.axis_index("c")
    s = lax.axis_index("s")
    wid = c * NS + s
    _fill1d(stage, ROWS_PT, 0.0)
    pltpu.sync_copy(stage, deg_sh.at[pl.ds(s * ROWS_PT, ROWS_PT)])
    plsc.subcore_barrier()
    _fill1d(ones_v, CHUNK, 1.0)
    pltpu.sync_copy(dst3d.at[wid], dstv)

    def body(j, _):
        pltpu.sync_copy(ones_v, deg_sh.at[dstv.at[j]], add=True)
        return 0
    lax.fori_loop(0, cpwd, body, 0)
    plsc.subcore_barrier()
    pltpu.sync_copy(deg_sh.at[pl.ds(s * ROWS_PT, ROWS_PT)], stage)
    pltpu.sync_copy(stage, deg_out.at[c, 0, pl.ds(s * ROWS_PT, ROWS_PT)])


def _sc_deg(dst3d_deg, cpwd):
    kern = pl.kernel(
        functools.partial(_deg_body, cpwd),
        out_type=jax.ShapeDtypeStruct((NC, 1, R_PAD), jnp.float32),
        mesh=plsc.VectorSubcoreMesh(core_axis_name="c", subcore_axis_name="s"),
        compiler_params=pltpu.CompilerParams(use_tc_tiling_on_sc=False),
        scratch_types=[
            pltpu.VMEM((cpwd, CHUNK), jnp.int32),
            pltpu.VMEM((CHUNK,), jnp.float32),
            pltpu.VMEM((ROWS_PT,), jnp.float32),
            pltpu.VMEM_SHARED((R_PAD,), jnp.float32),
        ],
    )
    return kern(dst3d_deg)


def _agg_body(cpt, d, staged, hs2, src3d, dst3d, out,
              srcv, dstv, buf0, buf1, buf2, buf3, acc_sh,
              gs0, gs1, gs2_, gs3, ss0, ss1, ss2, ss3, *maybe_tab):
    c = lax.axis_index("c")
    s = lax.axis_index("s")
    _fill2d(buf0, CHUNK, d, 0.0)
    for r in range(ROWS_PT // CHUNK):
        pltpu.sync_copy(buf0, acc_sh.at[pl.ds(s * ROWS_PT + r * CHUNK, CHUNK)])
    if staged:
        tab = maybe_tab[0]
        pltpu.sync_copy(hs2.at[c, pl.ds(s * OUT_PT, OUT_PT)],
                        tab.at[pl.ds(s * OUT_PT, OUT_PT)])
    else:
        tab = hs2.at[c]
    plsc.subcore_barrier()
    pltpu.sync_copy(src3d.at[s], srcv)
    pltpu.sync_copy(dst3d.at[s], dstv)

    bufs = (buf0, buf1, buf2, buf3)
    gsem = (gs0, gs1, gs2_, gs3)
    ssem = (ss0, ss1, ss2, ss3)
    for k in range(4):
        pltpu.async_copy(tab.at[srcv.at[k]], bufs[k], gsem[k])

    def group(jj, _):
        for k in range(4):
            j = 4 * jj + k
            pltpu.make_async_copy(tab.at[srcv.at[0]], bufs[k], gsem[k]).wait()
            pltpu.async_copy(bufs[k], acc_sh.at[dstv.at[j]], ssem[k], add=True)
            jn = jnp.where(j + 4 < cpt, j + 4, 0)
            pltpu.make_async_copy(bufs[k], acc_sh.at[dstv.at[0]],
                                  ssem[k]).wait()
            pltpu.async_copy(tab.at[srcv.at[jn]], bufs[k], gsem[k])
        return 0
    lax.fori_loop(0, cpt // 4, group, 0)
    for k in range(4):
        pltpu.make_async_copy(tab.at[srcv.at[0]], bufs[k], gsem[k]).wait()

    plsc.subcore_barrier()
    for r0, nr in ((0, CHUNK), (CHUNK, CHUNK), (2 * CHUNK, CHUNK),
                   (3 * CHUNK, CHUNK), (4 * CHUNK, OUT_PT - 4 * CHUNK)):
        row0 = s * OUT_PT + r0
        pltpu.sync_copy(acc_sh.at[pl.ds(row0, nr)], buf0.at[pl.ds(0, nr)])
        pltpu.sync_copy(buf0.at[pl.ds(0, nr)], out.at[c, pl.ds(row0, nr)])


def _sc_agg(hs2, src3d, dst3d, cpt, d, staged):
    scratch = [
        pltpu.VMEM((cpt, CHUNK), jnp.int32),
        pltpu.VMEM((cpt, CHUNK), jnp.int32),
        pltpu.VMEM((CHUNK, d), jnp.bfloat16),
        pltpu.VMEM((CHUNK, d), jnp.bfloat16),
        pltpu.VMEM((CHUNK, d), jnp.bfloat16),
        pltpu.VMEM((CHUNK, d), jnp.bfloat16),
        pltpu.VMEM_SHARED((R_PAD, d), jnp.bfloat16),
        pltpu.SemaphoreType.DMA,
        pltpu.SemaphoreType.DMA,
        pltpu.SemaphoreType.DMA,
        pltpu.SemaphoreType.DMA,
        pltpu.SemaphoreType.DMA,
        pltpu.SemaphoreType.DMA,
        pltpu.SemaphoreType.DMA,
        pltpu.SemaphoreType.DMA,
    ]
    if staged:
        scratch.append(pltpu.VMEM_SHARED((N_NODES, d), jnp.bfloat16))
    kern = pl.kernel(
        functools.partial(_agg_body, cpt, d, staged),
        out_type=jax.ShapeDtypeStruct((NC, N_NODES, d), jnp.bfloat16),
        mesh=plsc.VectorSubcoreMesh(core_axis_name="c", subcore_axis_name="s"),
        compiler_params=pltpu.CompilerParams(use_tc_tiling_on_sc=False),
        scratch_types=scratch,
    )
    return kern(hs2, src3d, dst3d)


def _mm1_body(x_ref, w_ref, deg_ref, o_ref):
    dinv = lax.rsqrt(deg_ref[0] + deg_ref[1] + 1.0)
    h = jnp.dot(x_ref[...], w_ref[...], preferred_element_type=jnp.float32)
    hs = (h * dinv).astype(jnp.bfloat16)
    o_ref[0] = hs[:, :64]
    o_ref[1] = hs[:, 64:]


def _mm2_body(p_ref, h_ref, deg_ref, b_ref, w_ref, o_ref):
    dinv = lax.rsqrt(deg_ref[0] + deg_ref[1] + 1.0)
    agg = jnp.concatenate([p_ref[0] + h_ref[0], p_ref[1] + h_ref[1]],
                          axis=1).astype(jnp.float32)
    h1 = jnp.maximum(agg * dinv + b_ref[...], 0.0)
    g = ((jnp.dot(h1, w_ref[...], preferred_element_type=jnp.float32) * dinv)
         .astype(jnp.bfloat16))
    o_ref[0] = g[:, :32]
    o_ref[1] = g[:, 32:]


def _fin_body(p_ref, g_ref, deg_ref, b_ref, o_ref):
    dinv = lax.rsqrt(deg_ref[0] + deg_ref[1] + 1.0)
    agg = jnp.concatenate([p_ref[0] + g_ref[0], p_ref[1] + g_ref[1]],
                          axis=1).astype(jnp.float32)
    o_ref[...] = agg * dinv + b_ref[...]


def _tc_mm1(x, w1, deg3d):
    return pl.pallas_call(
        _mm1_body,
        grid=(N_NODES // RB,),
        in_specs=[
            pl.BlockSpec((RB, 128), lambda i: (i, 0)),
            pl.BlockSpec((128, 128), lambda i: (0, 0)),
            pl.BlockSpec((NC, RB, 1), lambda i: (0, i, 0)),
        ],
        out_specs=pl.BlockSpec((NC, RB, 64), lambda i: (0, i, 0)),
        out_shape=jax.ShapeDtypeStruct((NC, N_NODES, 64), jnp.bfloat16),
    )(x, w1, deg3d)


def _tc_mm2(part1, hs2, deg3d, b1, w2):
    return pl.pallas_call(
        _mm2_body,
        grid=(N_NODES // RB,),
        in_specs=[
            pl.BlockSpec((NC, RB, 64), lambda i: (0, i, 0)),
            pl.BlockSpec((NC, RB, 64), lambda i: (0, i, 0)),
            pl.BlockSpec((NC, RB, 1), lambda i: (0, i, 0)),
            pl.BlockSpec((1, 128), lambda i: (0, 0)),
            pl.BlockSpec((128, 64), lambda i: (0, 0)),
        ],
        out_specs=pl.BlockSpec((NC, RB, 32), lambda i: (0, i, 0)),
        out_shape=jax.ShapeDtypeStruct((NC, N_NODES, 32), jnp.bfloat16),
    )(part1, hs2, deg3d, b1, w2)


def _tc_fin(part2, gs2, deg3d, b2):
    return pl.pallas_call(
        _fin_body,
        grid=(N_NODES // RB,),
        in_specs=[
            pl.BlockSpec((NC, RB, 32), lambda i: (0, i, 0)),
            pl.BlockSpec((NC, RB, 32), lambda i: (0, i, 0)),
            pl.BlockSpec((NC, RB, 1), lambda i: (0, i, 0)),
            pl.BlockSpec((1, 64), lambda i: (0, 0)),
        ],
        out_specs=pl.BlockSpec((RB, 64), lambda i: (i, 0)),
        out_shape=jax.ShapeDtypeStruct((N_NODES, 64), jnp.float32),
    )(part2, gs2, deg3d, b2)


def kernel(x, edge_index, W1, b1, W2, b2):
    n = x.shape[0]
    e = edge_index.shape[1]
    cpt = -(-e // (NS * CHUNK))
    cpt += (-cpt) % 4
    e_padded = NS * cpt * CHUNK
    cpwd = cpt // 2
    npad = e_padded - e

    pad_src = jnp.arange(npad, dtype=jnp.int32) % n
    pad_dst = n + jnp.arange(npad, dtype=jnp.int32) % (R_PAD - n)
    src = jnp.concatenate([edge_index[0], pad_src])
    dst = jnp.concatenate([edge_index[1], pad_dst])
    src3d = src.reshape(NS, cpt, CHUNK)
    dst3d = dst.reshape(NS, cpt, CHUNK)
    dst3d_deg = dst.reshape(NW, cpwd, CHUNK)

    deg3d = _sc_deg(dst3d_deg, cpwd).reshape(NC, R_PAD, 1)
    hs2 = _tc_mm1(x, W1, deg3d)
    part1 = _sc_agg(hs2, src3d, dst3d, cpt, 64, True)
    gs2 = _tc_mm2(part1, hs2, deg3d, b1.reshape(1, 128), W2)
    part2 = _sc_agg(gs2, src3d, dst3d, cpt, 32, True)
    out = _tc_fin(part2, gs2, deg3d, b2.reshape(1, 64))
    return out
